# Initial kernel scaffold; baseline (speedup 1.0000x reference)
import jax
import jax.numpy as jnp
from jax.experimental import pallas as pl


def kernel(x, edge_index, W1, b1, W2, b2):
    raise NotImplementedError("write your pallas kernel here")


# speedup vs baseline: 29.3201x; 29.3201x over previous
import functools

import jax
import jax.numpy as jnp
from jax import lax
from jax.experimental import pallas as pl
from jax.experimental.pallas import tpu as pltpu
from jax.experimental.pallas import tpu_sc as plsc

_NC = 2
_NS = 16
_L = 16
_NW = _NC * _NS
_ROW = 128


def _sc_mesh():
    return plsc.VectorSubcoreMesh(core_axis_name="c", subcore_axis_name="s")


_SC_PARAMS = pltpu.CompilerParams(use_tc_tiling_on_sc=False)


def _sc_degree(dst_rows, n_pad):
    rows = dst_rows.shape[0]
    rpw = rows // _NW
    npt = n_pad // _NS

    @functools.partial(
        pl.kernel,
        out_type=jax.ShapeDtypeStruct((_NC, n_pad, _L), jnp.float32),
        mesh=_sc_mesh(),
        compiler_params=_SC_PARAMS,
        scratch_types=[
            pltpu.VMEM((rpw, _ROW), jnp.int32),
            pltpu.VMEM((_ROW, _L), jnp.float32),
            pltpu.VMEM((npt, _L), jnp.float32),
            pltpu.VMEM_SHARED((n_pad, _L), jnp.float32),
        ],
    )
    def deg_kernel(dst_hbm, out_hbm, dstv, onesv, zv, acc):
        c = lax.axis_index("c")
        s = lax.axis_index("s")
        wid = c * _NS + s

        @pl.loop(0, npt)
        def _(i):
            zv[i] = jnp.zeros((_L,), jnp.float32)

        @pl.loop(0, _ROW)
        def _(i):
            onesv[i] = jnp.ones((_L,), jnp.float32)

        pltpu.sync_copy(zv, acc.at[pl.ds(s * npt, npt)])
        pltpu.sync_copy(dst_hbm.at[pl.ds(wid * rpw, rpw)], dstv)
        plsc.subcore_barrier()

        @pl.loop(0, rpw)
        def _(r):
            pltpu.sync_copy(onesv, acc.at[dstv.at[r]], add=True)

        plsc.subcore_barrier()
        pltpu.sync_copy(acc.at[pl.ds(s * npt, npt)],
                        out_hbm.at[c, pl.ds(s * npt, npt)])

    return deg_kernel(dst_rows)


def _sc_aggregate(hs, src_rows, dst_rows):
    n_pad = hs.shape[0]
    rows = src_rows.shape[0]
    rpw = rows // _NW
    npt = n_pad // _NS

    @functools.partial(
        pl.kernel,
        out_type=jax.ShapeDtypeStruct((_NC, n_pad, _L), jnp.float32),
        mesh=_sc_mesh(),
        compiler_params=_SC_PARAMS,
        scratch_types=[
            pltpu.VMEM((rpw, _ROW), jnp.int32),
            pltpu.VMEM((rpw, _ROW), jnp.int32),
            pltpu.VMEM((_ROW, _L), jnp.float32),
            pltpu.VMEM((npt, _L), jnp.float32),
            pltpu.VMEM_SHARED((n_pad, _L), jnp.float32),
        ],
    )
    def agg_kernel(hs_hbm, src_hbm, dst_hbm, out_hbm,
                   srcv, dstv, rowsv, zv, acc):
        c = lax.axis_index("c")
        s = lax.axis_index("s")
        wid = c * _NS + s

        @pl.loop(0, npt)
        def _(i):
            zv[i] = jnp.zeros((_L,), jnp.float32)

        pltpu.sync_copy(zv, acc.at[pl.ds(s * npt, npt)])
        pltpu.sync_copy(src_hbm.at[pl.ds(wid * rpw, rpw)], srcv)
        pltpu.sync_copy(dst_hbm.at[pl.ds(wid * rpw, rpw)], dstv)
        plsc.subcore_barrier()

        @pl.loop(0, rpw)
        def _(r):
            pltpu.sync_copy(hs_hbm.at[srcv.at[r]], rowsv)
            pltpu.sync_copy(rowsv, acc.at[dstv.at[r]], add=True)

        plsc.subcore_barrier()
        pltpu.sync_copy(acc.at[pl.ds(s * npt, npt)],
                        out_hbm.at[c, pl.ds(s * npt, npt)])

    return agg_kernel(hs, src_rows, dst_rows)


def _tc_mm1(x, w, n_pad):
    n, _ = x.shape
    d_out = w.shape[1]

    def body(x_ref, w_ref, o_ref):
        o_ref[...] = jnp.zeros((n_pad, d_out), jnp.float32)
        o_ref[0:n, :] = jnp.dot(x_ref[...], w_ref[...],
                                preferred_element_type=jnp.float32)

    return pl.pallas_call(
        body,
        out_shape=jax.ShapeDtypeStruct((n_pad, d_out), jnp.float32),
    )(x, w)


def _tc_scale(h, degp):
    def body(h_ref, d_ref, o_ref):
        dinv = lax.rsqrt(d_ref[0] + d_ref[1] + 1.0)
        o_ref[...] = h_ref[...] * dinv

    return pl.pallas_call(
        body,
        out_shape=jax.ShapeDtypeStruct(h.shape, jnp.float32),
    )(h, degp)


def _tc_post1(acc, hs1, degp, b1):
    def body(a_ref, h_ref, d_ref, b_ref, o_ref):
        dinv = lax.rsqrt(d_ref[0] + d_ref[1] + 1.0)
        t = (a_ref[0] + a_ref[1] + h_ref[...]) * dinv + b_ref[...]
        o_ref[...] = jnp.maximum(t, 0.0) * dinv

    return pl.pallas_call(
        body,
        out_shape=jax.ShapeDtypeStruct(hs1.shape, jnp.float32),
    )(acc, hs1, degp, b1)


def _tc_post2(acc, hs2, degp, w2, b2, n):
    d_out = w2.shape[1]

    def body(a_ref, h_ref, d_ref, w_ref, b_ref, o_ref):
        dinv = lax.rsqrt(d_ref[0, 0:n] + d_ref[1, 0:n] + 1.0)
        z = (a_ref[0, 0:n] + a_ref[1, 0:n] + h_ref[0:n]) * dinv
        y = jnp.dot(z, w_ref[...], preferred_element_type=jnp.float32)
        y = y + b_ref[...]
        m = jnp.max(y, axis=1, keepdims=True)
        e = jnp.exp(y - m)
        o_ref[...] = (y - m) - jnp.log(jnp.sum(e, axis=1, keepdims=True))

    return pl.pallas_call(
        body,
        out_shape=jax.ShapeDtypeStruct((n, d_out), jnp.float32),
    )(acc, hs2, degp, w2, b2)


def kernel(x, edge_index, W1, b1, W2, b2):
    n = x.shape[0]
    e = edge_index.shape[1]
    n_pad = ((n + 1 + _NS * 8 - 1) // (_NS * 8)) * (_NS * 8)
    e_quant = _NW * _ROW * 8
    e_pad = ((e + e_quant - 1) // e_quant) * e_quant

    src = edge_index[0]
    dst = edge_index[1]
    pad = jnp.full((e_pad - e,), n, dtype=jnp.int32)
    src_rows = jnp.concatenate([src, pad]).reshape(e_pad // _ROW, _ROW)
    dst_rows = jnp.concatenate([dst, pad]).reshape(e_pad // _ROW, _ROW)

    degp = _sc_degree(dst_rows, n_pad)
    h1 = _tc_mm1(x, W1, n_pad)
    hs1 = _tc_scale(h1, degp)
    acc1 = _sc_aggregate(hs1, src_rows, dst_rows)
    hs2 = _tc_post1(acc1, hs1, degp, b1)
    acc2 = _sc_aggregate(hs2, src_rows, dst_rows)
    return _tc_post2(acc2, hs2, degp, W2, b2, n)


# speedup vs baseline: 32.7775x; 1.1179x over previous
import functools

import jax
import jax.numpy as jnp
from jax import lax
from jax.experimental import pallas as pl
from jax.experimental.pallas import tpu as pltpu
from jax.experimental.pallas import tpu_sc as plsc

_NC = 2
_NS = 16
_L = 16
_NW = _NC * _NS
_ROW = 128
_RPW = 80
_K = 8

_E_PAD = _NW * _RPW * _ROW


def _sc_mesh():
    return plsc.VectorSubcoreMesh(core_axis_name="c", subcore_axis_name="s")


_SC_PARAMS = pltpu.CompilerParams(use_tc_tiling_on_sc=False)


def _sc_degree(dst_rows, ones_hbm, zeros_hbm, n_pad):
    npt = n_pad // _NS

    @functools.partial(
        pl.kernel,
        out_type=jax.ShapeDtypeStruct((_NC, n_pad, _L), jnp.float32),
        mesh=_sc_mesh(),
        compiler_params=_SC_PARAMS,
        scratch_types=[
            pltpu.VMEM((_RPW, _ROW), jnp.int32),
            pltpu.VMEM((_ROW, _L), jnp.float32),
            pltpu.VMEM_SHARED((n_pad, _L), jnp.float32),
            pltpu.SemaphoreType.DMA,
        ],
    )
    def deg_kernel(dst_hbm, ones_h, zeros_h, out_hbm, dstv, onesv, acc, sem):
        c = lax.axis_index("c")
        s = lax.axis_index("s")
        wid = c * _NS + s

        pltpu.sync_copy(ones_h, onesv)
        pltpu.sync_copy(zeros_h, acc.at[pl.ds(s * npt, npt)])
        pltpu.sync_copy(dst_hbm.at[wid], dstv)
        plsc.subcore_barrier()

        @pl.loop(0, _RPW, step=_K)
        def _chunk(r0):
            copies = [
                pltpu.async_copy(onesv, acc.at[dstv.at[r0 + j]], sem,
                                 add=True)
                for j in range(_K)
            ]
            for cp in copies:
                cp.wait()

        plsc.subcore_barrier()
        pltpu.sync_copy(acc.at[pl.ds(s * npt, npt)],
                        out_hbm.at[c, pl.ds(s * npt, npt)])

    return deg_kernel(dst_rows, ones_hbm, zeros_hbm)


def _sc_aggregate(hs, src_rows, dst_rows, zeros_hbm):
    n_pad = hs.shape[0]
    npt = n_pad // _NS

    @functools.partial(
        pl.kernel,
        out_type=jax.ShapeDtypeStruct((_NC, n_pad, _L), jnp.float32),
        mesh=_sc_mesh(),
        compiler_params=_SC_PARAMS,
        scratch_types=[
            pltpu.VMEM((_RPW, _ROW), jnp.int32),
            pltpu.VMEM((_RPW, _ROW), jnp.int32),
            pltpu.VMEM((_K, _ROW, _L), jnp.float32),
            pltpu.VMEM_SHARED((n_pad, _L), jnp.float32),
            pltpu.SemaphoreType.DMA,
            pltpu.SemaphoreType.DMA,
        ],
    )
    def agg_kernel(hs_hbm, src_hbm, dst_hbm, zeros_h, out_hbm,
                   srcv, dstv, rows, acc, gsem, ssem):
        c = lax.axis_index("c")
        s = lax.axis_index("s")
        wid = c * _NS + s

        pltpu.sync_copy(zeros_h, acc.at[pl.ds(s * npt, npt)])
        pltpu.sync_copy(src_hbm.at[wid], srcv)
        pltpu.sync_copy(dst_hbm.at[wid], dstv)
        plsc.subcore_barrier()

        @pl.loop(0, _RPW, step=_K)
        def _chunk(r0):
            gathers = [
                pltpu.async_copy(hs_hbm.at[srcv.at[r0 + j]], rows.at[j],
                                 gsem)
                for j in range(_K)
            ]
            for cp in gathers:
                cp.wait()
            scatters = [
                pltpu.async_copy(rows.at[j], acc.at[dstv.at[r0 + j]], ssem,
                                 add=True)
                for j in range(_K)
            ]
            for cp in scatters:
                cp.wait()

        plsc.subcore_barrier()
        pltpu.sync_copy(acc.at[pl.ds(s * npt, npt)],
                        out_hbm.at[c, pl.ds(s * npt, npt)])

    return agg_kernel(hs, src_rows, dst_rows, zeros_hbm)


def _tc_mm1(x, w, n_pad):
    n, _ = x.shape
    d_out = w.shape[1]

    def body(x_ref, w_ref, o_ref):
        o_ref[...] = jnp.zeros((n_pad, d_out), jnp.float32)
        o_ref[0:n, :] = jnp.dot(x_ref[...], w_ref[...],
                                preferred_element_type=jnp.float32)

    return pl.pallas_call(
        body,
        out_shape=jax.ShapeDtypeStruct((n_pad, d_out), jnp.float32),
    )(x, w)


def _tc_scale(h, degp):
    def body(h_ref, d_ref, o_ref):
        dinv = lax.rsqrt(d_ref[0] + d_ref[1] + 1.0)
        o_ref[...] = h_ref[...] * dinv

    return pl.pallas_call(
        body,
        out_shape=jax.ShapeDtypeStruct(h.shape, jnp.float32),
    )(h, degp)


def _tc_post1(acc, hs1, degp, b1):
    def body(a_ref, h_ref, d_ref, b_ref, o_ref):
        dinv = lax.rsqrt(d_ref[0] + d_ref[1] + 1.0)
        t = (a_ref[0] + a_ref[1] + h_ref[...]) * dinv + b_ref[...]
        o_ref[...] = jnp.maximum(t, 0.0) * dinv

    return pl.pallas_call(
        body,
        out_shape=jax.ShapeDtypeStruct(hs1.shape, jnp.float32),
    )(acc, hs1, degp, b1)


def _tc_post2(acc, hs2, degp, w2, b2, n):
    d_out = w2.shape[1]

    def body(a_ref, h_ref, d_ref, w_ref, b_ref, o_ref):
        dinv = lax.rsqrt(d_ref[0, 0:n] + d_ref[1, 0:n] + 1.0)
        z = (a_ref[0, 0:n] + a_ref[1, 0:n] + h_ref[0:n]) * dinv
        y = jnp.dot(z, w_ref[...], preferred_element_type=jnp.float32)
        y = y + b_ref[...]
        m = jnp.max(y, axis=1, keepdims=True)
        e = jnp.exp(y - m)
        o_ref[...] = (y - m) - jnp.log(jnp.sum(e, axis=1, keepdims=True))

    return pl.pallas_call(
        body,
        out_shape=jax.ShapeDtypeStruct((n, d_out), jnp.float32),
    )(acc, hs2, degp, w2, b2)


def kernel(x, edge_index, W1, b1, W2, b2):
    n = x.shape[0]
    e = edge_index.shape[1]
    n_pad = ((n + 1 + _NS * 8 - 1) // (_NS * 8)) * (_NS * 8)
    assert _E_PAD >= e

    src = edge_index[0]
    dst = edge_index[1]
    pad = jnp.full((_E_PAD - e,), n, dtype=jnp.int32)
    row_shape = (_NW, _RPW, _ROW)
    src_rows = jnp.concatenate([src, pad]).reshape(row_shape)
    dst_rows = jnp.concatenate([dst, pad]).reshape(row_shape)
    ones_hbm = jnp.ones((_ROW, _L), jnp.float32)
    zeros_hbm = jnp.zeros((n_pad // _NS, _L), jnp.float32)

    degp = _sc_degree(dst_rows, ones_hbm, zeros_hbm, n_pad)
    h1 = _tc_mm1(x, W1, n_pad)
    hs1 = _tc_scale(h1, degp)
    acc1 = _sc_aggregate(hs1, src_rows, dst_rows, zeros_hbm)
    hs2 = _tc_post1(acc1, hs1, degp, b1)
    acc2 = _sc_aggregate(hs2, src_rows, dst_rows, zeros_hbm)
    return _tc_post2(acc2, hs2, degp, W2, b2, n)


# speedup vs baseline: 34.0251x; 1.0381x over previous
import functools

import jax
import jax.numpy as jnp
from jax import lax
from jax.experimental import pallas as pl
from jax.experimental.pallas import tpu as pltpu
from jax.experimental.pallas import tpu_sc as plsc

_NC = 2
_NS = 16
_L = 16
_NW = _NC * _NS
_ROW = 128
_RPW = 80
_K = 8

_E_PAD = _NW * _RPW * _ROW


def _sc_mesh():
    return plsc.VectorSubcoreMesh(core_axis_name="c", subcore_axis_name="s")


_SC_PARAMS = pltpu.CompilerParams(use_tc_tiling_on_sc=False)


def _sc_degree(dst_rows, ones_hbm, zeros_hbm, n_pad):
    npt = n_pad // _NS

    @functools.partial(
        pl.kernel,
        out_type=jax.ShapeDtypeStruct((_NC, n_pad, _L), jnp.float32),
        mesh=_sc_mesh(),
        compiler_params=_SC_PARAMS,
        scratch_types=[
            pltpu.VMEM((_RPW, _ROW), jnp.int32),
            pltpu.VMEM((_ROW, _L), jnp.float32),
            pltpu.VMEM_SHARED((n_pad, _L), jnp.float32),
            pltpu.SemaphoreType.DMA,
        ],
    )
    def deg_kernel(dst_hbm, ones_h, zeros_h, out_hbm, dstv, onesv, acc, sem):
        c = lax.axis_index("c")
        s = lax.axis_index("s")
        wid = c * _NS + s

        pltpu.sync_copy(ones_h, onesv)
        pltpu.sync_copy(zeros_h, acc.at[pl.ds(s * npt, npt)])
        pltpu.sync_copy(dst_hbm.at[wid], dstv)
        plsc.subcore_barrier()

        copies = [
            pltpu.async_copy(onesv, acc.at[dstv.at[r]], sem, add=True)
            for r in range(_RPW)
        ]
        for cp in copies:
            cp.wait()

        plsc.subcore_barrier()
        pltpu.sync_copy(acc.at[pl.ds(s * npt, npt)],
                        out_hbm.at[c, pl.ds(s * npt, npt)])

    return deg_kernel(dst_rows, ones_hbm, zeros_hbm)


def _sc_aggregate(hs, src_rows, dst_rows, zeros_hbm):
    n_pad = hs.shape[0]
    npt = n_pad // _NS

    @functools.partial(
        pl.kernel,
        out_type=jax.ShapeDtypeStruct((_NC, n_pad, _L), jnp.float32),
        mesh=_sc_mesh(),
        compiler_params=_SC_PARAMS,
        scratch_types=[
            pltpu.VMEM((_RPW, _ROW), jnp.int32),
            pltpu.VMEM((_RPW, _ROW), jnp.int32),
            pltpu.VMEM((2, _K, _ROW, _L), jnp.float32),
            pltpu.VMEM_SHARED((n_pad, _L), jnp.float32),
            pltpu.SemaphoreType.DMA,
            pltpu.SemaphoreType.DMA,
            pltpu.SemaphoreType.DMA,
            pltpu.SemaphoreType.DMA,
        ],
    )
    def agg_kernel(hs_hbm, src_hbm, dst_hbm, zeros_h, out_hbm,
                   srcv, dstv, rows, acc, gsem0, gsem1, ssem0, ssem1):
        c = lax.axis_index("c")
        s = lax.axis_index("s")
        wid = c * _NS + s
        gsems = (gsem0, gsem1)
        ssems = (ssem0, ssem1)

        pltpu.sync_copy(zeros_h, acc.at[pl.ds(s * npt, npt)])
        pltpu.sync_copy(src_hbm.at[wid], srcv)
        pltpu.sync_copy(dst_hbm.at[wid], dstv)
        plsc.subcore_barrier()

        n_chunks = _RPW // _K

        def fire_gathers(ch):
            h = ch % 2
            return [
                pltpu.async_copy(hs_hbm.at[srcv.at[ch * _K + j]],
                                 rows.at[h, j], gsems[h])
                for j in range(_K)
            ]

        def fire_scatters(ch):
            h = ch % 2
            return [
                pltpu.async_copy(rows.at[h, j], acc.at[dstv.at[ch * _K + j]],
                                 ssems[h], add=True)
                for j in range(_K)
            ]

        gathers = fire_gathers(0)
        scatters = [None] * n_chunks
        for ch in range(n_chunks):
            for cp in gathers:
                cp.wait()
            if ch + 1 < n_chunks:
                if ch >= 1:
                    for cp in scatters[ch - 1]:
                        cp.wait()
                gathers = fire_gathers(ch + 1)
            scatters[ch] = fire_scatters(ch)
        for ch in (n_chunks - 2, n_chunks - 1):
            for cp in scatters[ch]:
                cp.wait()

        plsc.subcore_barrier()
        pltpu.sync_copy(acc.at[pl.ds(s * npt, npt)],
                        out_hbm.at[c, pl.ds(s * npt, npt)])

    return agg_kernel(hs, src_rows, dst_rows, zeros_hbm)


def _tc_mm1(x, w, n_pad):
    n, _ = x.shape
    d_out = w.shape[1]

    def body(x_ref, w_ref, o_ref):
        o_ref[...] = jnp.zeros((n_pad, d_out), jnp.float32)
        o_ref[0:n, :] = jnp.dot(x_ref[...], w_ref[...],
                                preferred_element_type=jnp.float32)

    return pl.pallas_call(
        body,
        out_shape=jax.ShapeDtypeStruct((n_pad, d_out), jnp.float32),
    )(x, w)


def _tc_scale(h, degp):
    def body(h_ref, d_ref, o_ref):
        dinv = lax.rsqrt(d_ref[0] + d_ref[1] + 1.0)
        o_ref[...] = h_ref[...] * dinv

    return pl.pallas_call(
        body,
        out_shape=jax.ShapeDtypeStruct(h.shape, jnp.float32),
    )(h, degp)


def _tc_post1(acc, hs1, degp, b1):
    def body(a_ref, h_ref, d_ref, b_ref, o_ref):
        dinv = lax.rsqrt(d_ref[0] + d_ref[1] + 1.0)
        t = (a_ref[0] + a_ref[1] + h_ref[...]) * dinv + b_ref[...]
        o_ref[...] = jnp.maximum(t, 0.0) * dinv

    return pl.pallas_call(
        body,
        out_shape=jax.ShapeDtypeStruct(hs1.shape, jnp.float32),
    )(acc, hs1, degp, b1)


def _tc_post2(acc, hs2, degp, w2, b2, n):
    d_out = w2.shape[1]

    def body(a_ref, h_ref, d_ref, w_ref, b_ref, o_ref):
        dinv = lax.rsqrt(d_ref[0, 0:n] + d_ref[1, 0:n] + 1.0)
        z = (a_ref[0, 0:n] + a_ref[1, 0:n] + h_ref[0:n]) * dinv
        y = jnp.dot(z, w_ref[...], preferred_element_type=jnp.float32)
        y = y + b_ref[...]
        m = jnp.max(y, axis=1, keepdims=True)
        e = jnp.exp(y - m)
        o_ref[...] = (y - m) - jnp.log(jnp.sum(e, axis=1, keepdims=True))

    return pl.pallas_call(
        body,
        out_shape=jax.ShapeDtypeStruct((n, d_out), jnp.float32),
    )(acc, hs2, degp, w2, b2)


def kernel(x, edge_index, W1, b1, W2, b2):
    n = x.shape[0]
    e = edge_index.shape[1]
    n_pad = ((n + 1 + _NS * 8 - 1) // (_NS * 8)) * (_NS * 8)
    assert _E_PAD >= e

    src = edge_index[0]
    dst = edge_index[1]
    pad = jnp.full((_E_PAD - e,), n, dtype=jnp.int32)
    row_shape = (_NW, _RPW, _ROW)
    src_rows = jnp.concatenate([src, pad]).reshape(row_shape)
    dst_rows = jnp.concatenate([dst, pad]).reshape(row_shape)
    ones_hbm = jnp.ones((_ROW, _L), jnp.float32)
    zeros_hbm = jnp.zeros((n_pad // _NS, _L), jnp.float32)

    degp = _sc_degree(dst_rows, ones_hbm, zeros_hbm, n_pad)
    h1 = _tc_mm1(x, W1, n_pad)
    hs1 = _tc_scale(h1, degp)
    acc1 = _sc_aggregate(hs1, src_rows, dst_rows, zeros_hbm)
    hs2 = _tc_post1(acc1, hs1, degp, b1)
    acc2 = _sc_aggregate(hs2, src_rows, dst_rows, zeros_hbm)
    return _tc_post2(acc2, hs2, degp, W2, b2, n)


# speedup vs baseline: 50.6616x; 1.4889x over previous
import functools

import jax
import jax.numpy as jnp
from jax import lax
from jax.experimental import pallas as pl
from jax.experimental.pallas import tpu as pltpu
from jax.experimental.pallas import tpu_sc as plsc

_NC = 2
_NS = 16
_L = 16
_NW = _NC * _NS
_ROW = 128
_RPW = 80
_K = 8

_E_PAD = _NW * _RPW * _ROW


def _sc_mesh():
    return plsc.VectorSubcoreMesh(core_axis_name="c", subcore_axis_name="s")


_SC_PARAMS = pltpu.CompilerParams(use_tc_tiling_on_sc=False)


def _sc_degree(dst_rows, ones_hbm, zeros_hbm, n_pad):
    npt = n_pad // _NS

    @functools.partial(
        pl.kernel,
        out_type=jax.ShapeDtypeStruct((_NC, n_pad, _L), jnp.float32),
        mesh=_sc_mesh(),
        compiler_params=_SC_PARAMS,
        scratch_types=[
            pltpu.VMEM((_RPW, _ROW), jnp.int32),
            pltpu.VMEM((_ROW, _L), jnp.float32),
            pltpu.VMEM_SHARED((n_pad, _L), jnp.float32),
            pltpu.SemaphoreType.DMA,
        ],
    )
    def deg_kernel(dst_hbm, ones_h, zeros_h, out_hbm, dstv, onesv, acc, sem):
        c = lax.axis_index("c")
        s = lax.axis_index("s")
        wid = c * _NS + s

        pltpu.sync_copy(ones_h, onesv)
        pltpu.sync_copy(zeros_h, acc.at[pl.ds(s * npt, npt)])
        pltpu.sync_copy(dst_hbm.at[wid], dstv)
        plsc.subcore_barrier()

        copies = [
            pltpu.async_copy(onesv, acc.at[dstv.at[r]], sem, add=True)
            for r in range(_RPW)
        ]
        for cp in copies:
            cp.wait()

        plsc.subcore_barrier()
        pltpu.sync_copy(acc.at[pl.ds(s * npt, npt)],
                        out_hbm.at[c, pl.ds(s * npt, npt)])

    return deg_kernel(dst_rows, ones_hbm, zeros_hbm)


def _sc_aggregate(hs, src_rows, dst_rows, zeros_hbm):
    n_pad = hs.shape[0]
    npt = n_pad // _NS

    @functools.partial(
        pl.kernel,
        out_type=jax.ShapeDtypeStruct((_NC, n_pad, _L), jnp.float32),
        mesh=_sc_mesh(),
        compiler_params=_SC_PARAMS,
        scratch_types=[
            pltpu.VMEM((_RPW, _ROW), jnp.int32),
            pltpu.VMEM((_RPW, _ROW), jnp.int32),
            pltpu.VMEM((2, _K, _ROW, _L), jnp.float32),
            pltpu.VMEM_SHARED((n_pad, _L), jnp.float32),
            pltpu.VMEM_SHARED((n_pad, _L), jnp.float32),
            pltpu.SemaphoreType.DMA,
            pltpu.SemaphoreType.DMA,
            pltpu.SemaphoreType.DMA,
            pltpu.SemaphoreType.DMA,
        ],
    )
    def agg_kernel(hs_hbm, src_hbm, dst_hbm, zeros_h, out_hbm,
                   srcv, dstv, rows, acc, hs_s, gsem0, gsem1, ssem0, ssem1):
        c = lax.axis_index("c")
        s = lax.axis_index("s")
        wid = c * _NS + s
        gsems = (gsem0, gsem1)
        ssems = (ssem0, ssem1)

        pltpu.sync_copy(hs_hbm.at[pl.ds(s * npt, npt)],
                        hs_s.at[pl.ds(s * npt, npt)])
        pltpu.sync_copy(zeros_h, acc.at[pl.ds(s * npt, npt)])
        pltpu.sync_copy(src_hbm.at[wid], srcv)
        pltpu.sync_copy(dst_hbm.at[wid], dstv)
        plsc.subcore_barrier()

        n_chunks = _RPW // _K

        def fire_gathers(ch):
            h = ch % 2
            return [
                pltpu.async_copy(hs_s.at[srcv.at[ch * _K + j]],
                                 rows.at[h, j], gsems[h])
                for j in range(_K)
            ]

        def fire_scatters(ch):
            h = ch % 2
            return [
                pltpu.async_copy(rows.at[h, j], acc.at[dstv.at[ch * _K + j]],
                                 ssems[h], add=True)
                for j in range(_K)
            ]

        gathers = fire_gathers(0)
        scatters = [None] * n_chunks
        for ch in range(n_chunks):
            for cp in gathers:
                cp.wait()
            if ch + 1 < n_chunks:
                if ch >= 1:
                    for cp in scatters[ch - 1]:
                        cp.wait()
                gathers = fire_gathers(ch + 1)
            scatters[ch] = fire_scatters(ch)
        for ch in (n_chunks - 2, n_chunks - 1):
            for cp in scatters[ch]:
                cp.wait()

        plsc.subcore_barrier()
        pltpu.sync_copy(acc.at[pl.ds(s * npt, npt)],
                        out_hbm.at[c, pl.ds(s * npt, npt)])

    return agg_kernel(hs, src_rows, dst_rows, zeros_hbm)


def _tc_mm1(x, w, n_pad):
    n, _ = x.shape
    d_out = w.shape[1]

    def body(x_ref, w_ref, o_ref):
        o_ref[...] = jnp.zeros((n_pad, d_out), jnp.float32)
        o_ref[0:n, :] = jnp.dot(x_ref[...], w_ref[...],
                                preferred_element_type=jnp.float32)

    return pl.pallas_call(
        body,
        out_shape=jax.ShapeDtypeStruct((n_pad, d_out), jnp.float32),
    )(x, w)


def _tc_scale(h, degp):
    def body(h_ref, d_ref, o_ref):
        dinv = lax.rsqrt(d_ref[0] + d_ref[1] + 1.0)
        o_ref[...] = h_ref[...] * dinv

    return pl.pallas_call(
        body,
        out_shape=jax.ShapeDtypeStruct(h.shape, jnp.float32),
    )(h, degp)


def _tc_post1(acc, hs1, degp, b1):
    def body(a_ref, h_ref, d_ref, b_ref, o_ref):
        dinv = lax.rsqrt(d_ref[0] + d_ref[1] + 1.0)
        t = (a_ref[0] + a_ref[1] + h_ref[...]) * dinv + b_ref[...]
        o_ref[...] = jnp.maximum(t, 0.0) * dinv

    return pl.pallas_call(
        body,
        out_shape=jax.ShapeDtypeStruct(hs1.shape, jnp.float32),
    )(acc, hs1, degp, b1)


def _tc_post2(acc, hs2, degp, w2, b2, n):
    d_out = w2.shape[1]

    def body(a_ref, h_ref, d_ref, w_ref, b_ref, o_ref):
        dinv = lax.rsqrt(d_ref[0, 0:n] + d_ref[1, 0:n] + 1.0)
        z = (a_ref[0, 0:n] + a_ref[1, 0:n] + h_ref[0:n]) * dinv
        y = jnp.dot(z, w_ref[...], preferred_element_type=jnp.float32)
        y = y + b_ref[...]
        m = jnp.max(y, axis=1, keepdims=True)
        e = jnp.exp(y - m)
        o_ref[...] = (y - m) - jnp.log(jnp.sum(e, axis=1, keepdims=True))

    return pl.pallas_call(
        body,
        out_shape=jax.ShapeDtypeStruct((n, d_out), jnp.float32),
    )(acc, hs2, degp, w2, b2)


def kernel(x, edge_index, W1, b1, W2, b2):
    n = x.shape[0]
    e = edge_index.shape[1]
    n_pad = ((n + 1 + _NS * 8 - 1) // (_NS * 8)) * (_NS * 8)
    assert _E_PAD >= e

    src = edge_index[0]
    dst = edge_index[1]
    pad = jnp.full((_E_PAD - e,), n, dtype=jnp.int32)
    row_shape = (_NW, _RPW, _ROW)
    src_rows = jnp.concatenate([src, pad]).reshape(row_shape)
    dst_rows = jnp.concatenate([dst, pad]).reshape(row_shape)
    ones_hbm = jnp.ones((_ROW, _L), jnp.float32)
    zeros_hbm = jnp.zeros((n_pad // _NS, _L), jnp.float32)

    degp = _sc_degree(dst_rows, ones_hbm, zeros_hbm, n_pad)
    h1 = _tc_mm1(x, W1, n_pad)
    hs1 = _tc_scale(h1, degp)
    acc1 = _sc_aggregate(hs1, src_rows, dst_rows, zeros_hbm)
    hs2 = _tc_post1(acc1, hs1, degp, b1)
    acc2 = _sc_aggregate(hs2, src_rows, dst_rows, zeros_hbm)
    return _tc_post2(acc2, hs2, degp, W2, b2, n)


# speedup vs baseline: 56.6292x; 1.1178x over previous
import functools

import jax
import jax.numpy as jnp
from jax import lax
from jax.experimental import pallas as pl
from jax.experimental.pallas import tpu as pltpu
from jax.experimental.pallas import tpu_sc as plsc

_NC = 2
_NS = 16
_L = 16
_NW = _NC * _NS
_ROW = 128
_RPW = 80
_K = 8

_E_PAD = _NW * _RPW * _ROW


def _sc_mesh():
    return plsc.VectorSubcoreMesh(core_axis_name="c", subcore_axis_name="s")


_SC_PARAMS = pltpu.CompilerParams(use_tc_tiling_on_sc=False)


def _edge_stream_pipeline(srcv, dstv, rows, hs_s, acc, gsems, ssems):
    n_chunks = _RPW // _K

    def fire_gathers(ch):
        h = ch % 2
        return [
            pltpu.async_copy(hs_s.at[srcv.at[ch * _K + j]],
                             rows.at[h, j], gsems[h])
            for j in range(_K)
        ]

    def fire_scatters(ch):
        h = ch % 2
        return [
            pltpu.async_copy(rows.at[h, j], acc.at[dstv.at[ch * _K + j]],
                             ssems[h], add=True)
            for j in range(_K)
        ]

    gathers = fire_gathers(0)
    scatters = [None] * n_chunks
    for ch in range(n_chunks):
        for cp in gathers:
            cp.wait()
        if ch + 1 < n_chunks:
            if ch >= 1:
                for cp in scatters[ch - 1]:
                    cp.wait()
            gathers = fire_gathers(ch + 1)
        scatters[ch] = fire_scatters(ch)
    for ch in (n_chunks - 2, n_chunks - 1):
        for cp in scatters[ch]:
            cp.wait()


def _sc_degree(dst_rows, ones_hbm, zeros_hbm, n_pad):
    npt = n_pad // _NS

    @functools.partial(
        pl.kernel,
        out_type=jax.ShapeDtypeStruct((_NC, n_pad, _L), jnp.float32),
        mesh=_sc_mesh(),
        compiler_params=_SC_PARAMS,
        scratch_types=[
            pltpu.VMEM((_RPW, _ROW), jnp.int32),
            pltpu.VMEM((_ROW, _L), jnp.float32),
            pltpu.VMEM_SHARED((n_pad, _L), jnp.float32),
            pltpu.SemaphoreType.DMA,
        ],
    )
    def deg_kernel(dst_hbm, ones_h, zeros_h, out_hbm, dstv, onesv, acc, sem):
        c = lax.axis_index("c")
        s = lax.axis_index("s")
        wid = c * _NS + s
        sl = pl.ds(s * npt, npt)

        pltpu.sync_copy(ones_h, onesv)
        pltpu.sync_copy(zeros_h, acc.at[sl])
        pltpu.sync_copy(dst_hbm.at[wid], dstv)
        plsc.subcore_barrier()

        copies = [
            pltpu.async_copy(onesv, acc.at[dstv.at[r]], sem, add=True)
            for r in range(_RPW)
        ]
        for cp in copies:
            cp.wait()

        plsc.subcore_barrier()
        pltpu.sync_copy(acc.at[sl], out_hbm.at[c, sl])

    return deg_kernel(dst_rows, ones_hbm, zeros_hbm)


def _sc_aggregate1(hs, src_rows, dst_rows, zeros_hbm):
    n_pad = hs.shape[0]
    npt = n_pad // _NS

    @functools.partial(
        pl.kernel,
        out_type=jax.ShapeDtypeStruct((_NC, n_pad, _L), jnp.float32),
        mesh=_sc_mesh(),
        compiler_params=_SC_PARAMS,
        scratch_types=[
            pltpu.VMEM((_RPW, _ROW), jnp.int32),
            pltpu.VMEM((_RPW, _ROW), jnp.int32),
            pltpu.VMEM((2, _K, _ROW, _L), jnp.float32),
            pltpu.VMEM_SHARED((n_pad, _L), jnp.float32),
            pltpu.VMEM_SHARED((n_pad, _L), jnp.float32),
            pltpu.SemaphoreType.DMA,
            pltpu.SemaphoreType.DMA,
            pltpu.SemaphoreType.DMA,
            pltpu.SemaphoreType.DMA,
        ],
    )
    def agg_kernel(hs_hbm, src_hbm, dst_hbm, zeros_h, out_hbm,
                   srcv, dstv, rows, acc, hs_s, gsem0, gsem1, ssem0, ssem1):
        c = lax.axis_index("c")
        s = lax.axis_index("s")
        wid = c * _NS + s
        sl = pl.ds(s * npt, npt)

        pltpu.sync_copy(hs_hbm.at[sl], hs_s.at[sl])

        @pl.when(c == 0)
        def _():
            pltpu.sync_copy(hs_hbm.at[sl], acc.at[sl])

        @pl.when(c != 0)
        def _():
            pltpu.sync_copy(zeros_h, acc.at[sl])

        pltpu.sync_copy(src_hbm.at[wid], srcv)
        pltpu.sync_copy(dst_hbm.at[wid], dstv)
        plsc.subcore_barrier()

        _edge_stream_pipeline(srcv, dstv, rows, hs_s, acc,
                              (gsem0, gsem1), (ssem0, ssem1))

        plsc.subcore_barrier()
        pltpu.sync_copy(acc.at[sl], out_hbm.at[c, sl])

    return agg_kernel(hs, src_rows, dst_rows, zeros_hbm)


def _sc_aggregate2(dinvb, acc1, b1_2d, src_rows, dst_rows, zeros_hbm):
    n_pad = dinvb.shape[0]
    npt = n_pad // _NS

    @functools.partial(
        pl.kernel,
        out_type=jax.ShapeDtypeStruct((_NC, n_pad, _L), jnp.float32),
        mesh=_sc_mesh(),
        compiler_params=_SC_PARAMS,
        scratch_types=[
            pltpu.VMEM((_RPW, _ROW), jnp.int32),
            pltpu.VMEM((_RPW, _ROW), jnp.int32),
            pltpu.VMEM((2, _K, _ROW, _L), jnp.float32),
            pltpu.VMEM((n_pad // _NS, _L), jnp.float32),
            pltpu.VMEM((n_pad // _NS, _L), jnp.float32),
            pltpu.VMEM((n_pad // _NS, _L), jnp.float32),
            pltpu.VMEM((n_pad // _NS, _L), jnp.float32),
            pltpu.VMEM((1, _L), jnp.float32),
            pltpu.VMEM_SHARED((n_pad, _L), jnp.float32),
            pltpu.VMEM_SHARED((n_pad, _L), jnp.float32),
            pltpu.SemaphoreType.DMA,
            pltpu.SemaphoreType.DMA,
            pltpu.SemaphoreType.DMA,
            pltpu.SemaphoreType.DMA,
        ],
    )
    def agg2_kernel(dinv_hbm, a_hbm, b1_hbm, src_hbm, dst_hbm, zeros_h,
                    out_hbm, srcv, dstv, rows, dinvv, a0v, a1v, hsv, b1v,
                    acc, hs_s, gsem0, gsem1, ssem0, ssem1):
        c = lax.axis_index("c")
        s = lax.axis_index("s")
        wid = c * _NS + s
        sl = pl.ds(s * npt, npt)

        pltpu.sync_copy(dinv_hbm.at[sl], dinvv)
        pltpu.sync_copy(a_hbm.at[0, sl], a0v)
        pltpu.sync_copy(a_hbm.at[1, sl], a1v)
        pltpu.sync_copy(b1_hbm, b1v)
        pltpu.sync_copy(src_hbm.at[wid], srcv)
        pltpu.sync_copy(dst_hbm.at[wid], dstv)

        b = b1v[0]

        def body(i, carry):
            dv = dinvv[i]
            t = (a0v[i] + a1v[i]) * dv + b
            hsv[i] = jnp.maximum(t, 0.0) * dv
            return carry

        lax.fori_loop(0, npt, body, 0)

        pltpu.sync_copy(hsv, hs_s.at[sl])

        @pl.when(c == 0)
        def _():
            pltpu.sync_copy(hsv, acc.at[sl])

        @pl.when(c != 0)
        def _():
            pltpu.sync_copy(zeros_h, acc.at[sl])

        plsc.subcore_barrier()

        _edge_stream_pipeline(srcv, dstv, rows, hs_s, acc,
                              (gsem0, gsem1), (ssem0, ssem1))

        plsc.subcore_barrier()
        pltpu.sync_copy(acc.at[sl], out_hbm.at[c, sl])

    return agg2_kernel(dinvb, acc1, b1_2d, src_rows, dst_rows, zeros_hbm)


def _tc_mm1(x, w, n_pad):
    n, _ = x.shape
    d_out = w.shape[1]

    def body(x_ref, w_ref, o_ref):
        o_ref[...] = jnp.zeros((n_pad, d_out), jnp.float32)
        o_ref[0:n, :] = jnp.dot(x_ref[...], w_ref[...],
                                preferred_element_type=jnp.float32)

    return pl.pallas_call(
        body,
        out_shape=jax.ShapeDtypeStruct((n_pad, d_out), jnp.float32),
    )(x, w)


def _tc_scale(h, degp):
    def body(h_ref, d_ref, hs_ref, di_ref):
        dinv = lax.rsqrt(d_ref[0] + d_ref[1] + 1.0)
        hs_ref[...] = h_ref[...] * dinv
        di_ref[...] = dinv

    return pl.pallas_call(
        body,
        out_shape=(jax.ShapeDtypeStruct(h.shape, jnp.float32),
                   jax.ShapeDtypeStruct(h.shape, jnp.float32)),
    )(h, degp)


def _tc_post2(acc, degp, w2, b2, n):
    d_out = w2.shape[1]

    def body(a_ref, d_ref, w_ref, b_ref, o_ref):
        dinv = lax.rsqrt(d_ref[0, 0:n] + d_ref[1, 0:n] + 1.0)
        z = (a_ref[0, 0:n] + a_ref[1, 0:n]) * dinv
        y = jnp.dot(z, w_ref[...], preferred_element_type=jnp.float32)
        y = y + b_ref[...]
        m = jnp.max(y, axis=1, keepdims=True)
        e = jnp.exp(y - m)
        o_ref[...] = (y - m) - jnp.log(jnp.sum(e, axis=1, keepdims=True))

    return pl.pallas_call(
        body,
        out_shape=jax.ShapeDtypeStruct((n, d_out), jnp.float32),
    )(acc, degp, w2, b2)


def kernel(x, edge_index, W1, b1, W2, b2):
    n = x.shape[0]
    e = edge_index.shape[1]
    n_pad = ((n + 1 + _NS * 8 - 1) // (_NS * 8)) * (_NS * 8)
    assert _E_PAD >= e

    src = edge_index[0]
    dst = edge_index[1]
    pad = jnp.full((_E_PAD - e,), n, dtype=jnp.int32)
    row_shape = (_NW, _RPW, _ROW)
    src_rows = jnp.concatenate([src, pad]).reshape(row_shape)
    dst_rows = jnp.concatenate([dst, pad]).reshape(row_shape)
    ones_hbm = jnp.ones((_ROW, _L), jnp.float32)
    zeros_hbm = jnp.zeros((n_pad // _NS, _L), jnp.float32)
    b1_2d = b1.reshape(1, _L)

    degp = _sc_degree(dst_rows, ones_hbm, zeros_hbm, n_pad)
    h1 = _tc_mm1(x, W1, n_pad)
    hs1, dinvb = _tc_scale(h1, degp)
    acc1 = _sc_aggregate1(hs1, src_rows, dst_rows, zeros_hbm)
    acc2 = _sc_aggregate2(dinvb, acc1, b1_2d, src_rows, dst_rows, zeros_hbm)
    return _tc_post2(acc2, degp, W2, b2, n)


# speedup vs baseline: 61.5045x; 1.0861x over previous
import functools

import jax
import jax.numpy as jnp
from jax import lax
from jax.experimental import pallas as pl
from jax.experimental.pallas import tpu as pltpu
from jax.experimental.pallas import tpu_sc as plsc

_NC = 2
_NS = 16
_L = 16
_NW = _NC * _NS
_ROW = 128
_RPW = 80
_K = 8

_E_PAD = _NW * _RPW * _ROW


def _sc_mesh():
    return plsc.VectorSubcoreMesh(core_axis_name="c", subcore_axis_name="s")


_SC_PARAMS = pltpu.CompilerParams(use_tc_tiling_on_sc=False)


def _edge_stream_pipeline(srcv, dstv, rows, hs_s, acc, gsems, ssems):
    n_chunks = _RPW // _K

    def fire_gathers(ch):
        h = ch % 2
        return [
            pltpu.async_copy(hs_s.at[srcv.at[ch * _K + j]],
                             rows.at[h, j], gsems[h])
            for j in range(_K)
        ]

    def fire_scatters(ch):
        h = ch % 2
        return [
            pltpu.async_copy(rows.at[h, j], acc.at[dstv.at[ch * _K + j]],
                             ssems[h], add=True)
            for j in range(_K)
        ]

    gathers = fire_gathers(0)
    scatters = [None] * n_chunks
    for ch in range(n_chunks):
        for cp in gathers:
            cp.wait()
        if ch + 1 < n_chunks:
            if ch >= 1:
                for cp in scatters[ch - 1]:
                    cp.wait()
            gathers = fire_gathers(ch + 1)
        scatters[ch] = fire_scatters(ch)
    for ch in (n_chunks - 2, n_chunks - 1):
        for cp in scatters[ch]:
            cp.wait()


def _sc_rsqrt(x):
    xi = lax.bitcast_convert_type(x, jnp.int32)
    yi = jnp.int32(0x5F3759DF) - lax.shift_right_logical(xi, 1)
    y = lax.bitcast_convert_type(yi, jnp.float32)
    for _ in range(3):
        y = y * (1.5 - 0.5 * x * y * y)
    return y


def _sc_degree(dst_rows, ones_hbm, zeros_hbm, n_pad):
    npt = n_pad // _NS

    @functools.partial(
        pl.kernel,
        out_type=jax.ShapeDtypeStruct((_NC, n_pad, _L), jnp.float32),
        mesh=_sc_mesh(),
        compiler_params=_SC_PARAMS,
        scratch_types=[
            pltpu.VMEM((_RPW, _ROW), jnp.int32),
            pltpu.VMEM((_ROW, _L), jnp.float32),
            pltpu.VMEM_SHARED((n_pad, _L), jnp.float32),
            pltpu.SemaphoreType.DMA,
        ],
    )
    def deg_kernel(dst_hbm, ones_h, zeros_h, out_hbm, dstv, onesv, acc, sem):
        c = lax.axis_index("c")
        s = lax.axis_index("s")
        wid = c * _NS + s
        sl = pl.ds(s * npt, npt)

        pltpu.sync_copy(ones_h, onesv)
        pltpu.sync_copy(zeros_h, acc.at[sl])
        pltpu.sync_copy(dst_hbm.at[wid], dstv)
        plsc.subcore_barrier()

        copies = [
            pltpu.async_copy(onesv, acc.at[dstv.at[r]], sem, add=True)
            for r in range(_RPW)
        ]
        for cp in copies:
            cp.wait()

        plsc.subcore_barrier()
        pltpu.sync_copy(acc.at[sl], out_hbm.at[c, sl])

    return deg_kernel(dst_rows, ones_hbm, zeros_hbm)


def _sc_aggregate1(h1, degp, src_rows, dst_rows, zeros_hbm):
    n_pad = h1.shape[0]
    npt = n_pad // _NS

    @functools.partial(
        pl.kernel,
        out_type=jax.ShapeDtypeStruct((_NC, n_pad, _L), jnp.float32),
        mesh=_sc_mesh(),
        compiler_params=_SC_PARAMS,
        scratch_types=[
            pltpu.VMEM((_RPW, _ROW), jnp.int32),
            pltpu.VMEM((_RPW, _ROW), jnp.int32),
            pltpu.VMEM((2, _K, _ROW, _L), jnp.float32),
            pltpu.VMEM((n_pad // _NS, _L), jnp.float32),
            pltpu.VMEM((n_pad // _NS, _L), jnp.float32),
            pltpu.VMEM((n_pad // _NS, _L), jnp.float32),
            pltpu.VMEM_SHARED((n_pad, _L), jnp.float32),
            pltpu.VMEM_SHARED((n_pad, _L), jnp.float32),
            pltpu.SemaphoreType.DMA,
            pltpu.SemaphoreType.DMA,
            pltpu.SemaphoreType.DMA,
            pltpu.SemaphoreType.DMA,
        ],
    )
    def agg_kernel(h_hbm, d_hbm, src_hbm, dst_hbm, zeros_h, out_hbm,
                   srcv, dstv, rows, hv, d0v, d1v, acc, hs_s,
                   gsem0, gsem1, ssem0, ssem1):
        c = lax.axis_index("c")
        s = lax.axis_index("s")
        wid = c * _NS + s
        sl = pl.ds(s * npt, npt)

        pltpu.sync_copy(h_hbm.at[sl], hv)
        pltpu.sync_copy(d_hbm.at[0, sl], d0v)
        pltpu.sync_copy(d_hbm.at[1, sl], d1v)
        pltpu.sync_copy(src_hbm.at[wid], srcv)
        pltpu.sync_copy(dst_hbm.at[wid], dstv)

        def body(i, carry):
            dv = _sc_rsqrt(d0v[i] + d1v[i] + 1.0)
            hv[i] = hv[i] * dv
            return carry

        lax.fori_loop(0, npt, body, 0)

        pltpu.sync_copy(hv, hs_s.at[sl])

        @pl.when(c == 0)
        def _():
            pltpu.sync_copy(hv, acc.at[sl])

        @pl.when(c != 0)
        def _():
            pltpu.sync_copy(zeros_h, acc.at[sl])

        plsc.subcore_barrier()

        _edge_stream_pipeline(srcv, dstv, rows, hs_s, acc,
                              (gsem0, gsem1), (ssem0, ssem1))

        plsc.subcore_barrier()
        pltpu.sync_copy(acc.at[sl], out_hbm.at[c, sl])

    return agg_kernel(h1, degp, src_rows, dst_rows, zeros_hbm)


def _sc_aggregate2(degp, acc1, b1_2d, src_rows, dst_rows, zeros_hbm):
    n_pad = acc1.shape[1]
    npt = n_pad // _NS

    @functools.partial(
        pl.kernel,
        out_type=jax.ShapeDtypeStruct((_NC, n_pad, _L), jnp.float32),
        mesh=_sc_mesh(),
        compiler_params=_SC_PARAMS,
        scratch_types=[
            pltpu.VMEM((_RPW, _ROW), jnp.int32),
            pltpu.VMEM((_RPW, _ROW), jnp.int32),
            pltpu.VMEM((2, _K, _ROW, _L), jnp.float32),
            pltpu.VMEM((n_pad // _NS, _L), jnp.float32),
            pltpu.VMEM((n_pad // _NS, _L), jnp.float32),
            pltpu.VMEM((n_pad // _NS, _L), jnp.float32),
            pltpu.VMEM((n_pad // _NS, _L), jnp.float32),
            pltpu.VMEM((n_pad // _NS, _L), jnp.float32),
            pltpu.VMEM((1, _L), jnp.float32),
            pltpu.VMEM_SHARED((n_pad, _L), jnp.float32),
            pltpu.VMEM_SHARED((n_pad, _L), jnp.float32),
            pltpu.SemaphoreType.DMA,
            pltpu.SemaphoreType.DMA,
            pltpu.SemaphoreType.DMA,
            pltpu.SemaphoreType.DMA,
        ],
    )
    def agg2_kernel(d_hbm, a_hbm, b1_hbm, src_hbm, dst_hbm, zeros_h,
                    out_hbm, srcv, dstv, rows, d0v, d1v, a0v, a1v, hsv,
                    b1v, acc, hs_s, gsem0, gsem1, ssem0, ssem1):
        c = lax.axis_index("c")
        s = lax.axis_index("s")
        wid = c * _NS + s
        sl = pl.ds(s * npt, npt)

        pltpu.sync_copy(d_hbm.at[0, sl], d0v)
        pltpu.sync_copy(d_hbm.at[1, sl], d1v)
        pltpu.sync_copy(a_hbm.at[0, sl], a0v)
        pltpu.sync_copy(a_hbm.at[1, sl], a1v)
        pltpu.sync_copy(b1_hbm, b1v)
        pltpu.sync_copy(src_hbm.at[wid], srcv)
        pltpu.sync_copy(dst_hbm.at[wid], dstv)

        b = b1v[0]

        def body(i, carry):
            dv = _sc_rsqrt(d0v[i] + d1v[i] + 1.0)
            t = (a0v[i] + a1v[i]) * dv + b
            hsv[i] = jnp.maximum(t, 0.0) * dv
            return carry

        lax.fori_loop(0, npt, body, 0)

        pltpu.sync_copy(hsv, hs_s.at[sl])

        @pl.when(c == 0)
        def _():
            pltpu.sync_copy(hsv, acc.at[sl])

        @pl.when(c != 0)
        def _():
            pltpu.sync_copy(zeros_h, acc.at[sl])

        plsc.subcore_barrier()

        _edge_stream_pipeline(srcv, dstv, rows, hs_s, acc,
                              (gsem0, gsem1), (ssem0, ssem1))

        plsc.subcore_barrier()
        pltpu.sync_copy(acc.at[sl], out_hbm.at[c, sl])

    return agg2_kernel(degp, acc1, b1_2d, src_rows, dst_rows, zeros_hbm)


def _tc_mm1(x, w, n_pad):
    n, _ = x.shape
    d_out = w.shape[1]

    def body(x_ref, w_ref, o_ref):
        o_ref[...] = jnp.zeros((n_pad, d_out), jnp.float32)
        o_ref[0:n, :] = jnp.dot(x_ref[...], w_ref[...],
                                preferred_element_type=jnp.float32)

    return pl.pallas_call(
        body,
        out_shape=jax.ShapeDtypeStruct((n_pad, d_out), jnp.float32),
    )(x, w)


def _tc_post2(acc, degp, w2, b2, n):
    d_out = w2.shape[1]

    def body(a_ref, d_ref, w_ref, b_ref, o_ref):
        dinv = lax.rsqrt(d_ref[0, 0:n] + d_ref[1, 0:n] + 1.0)
        z = (a_ref[0, 0:n] + a_ref[1, 0:n]) * dinv
        y = jnp.dot(z, w_ref[...], preferred_element_type=jnp.float32)
        y = y + b_ref[...]
        m = jnp.max(y, axis=1, keepdims=True)
        e = jnp.exp(y - m)
        o_ref[...] = (y - m) - jnp.log(jnp.sum(e, axis=1, keepdims=True))

    return pl.pallas_call(
        body,
        out_shape=jax.ShapeDtypeStruct((n, d_out), jnp.float32),
    )(acc, degp, w2, b2)


def kernel(x, edge_index, W1, b1, W2, b2):
    n = x.shape[0]
    e = edge_index.shape[1]
    n_pad = ((n + 1 + _NS * 8 - 1) // (_NS * 8)) * (_NS * 8)
    assert _E_PAD >= e

    src = edge_index[0]
    dst = edge_index[1]
    pad = jnp.full((_E_PAD - e,), n, dtype=jnp.int32)
    row_shape = (_NW, _RPW, _ROW)
    src_rows = jnp.concatenate([src, pad]).reshape(row_shape)
    dst_rows = jnp.concatenate([dst, pad]).reshape(row_shape)
    ones_hbm = jnp.ones((_ROW, _L), jnp.float32)
    zeros_hbm = jnp.zeros((n_pad // _NS, _L), jnp.float32)
    b1_2d = b1.reshape(1, _L)

    degp = _sc_degree(dst_rows, ones_hbm, zeros_hbm, n_pad)
    h1 = _tc_mm1(x, W1, n_pad)
    acc1 = _sc_aggregate1(h1, degp, src_rows, dst_rows, zeros_hbm)
    acc2 = _sc_aggregate2(degp, acc1, b1_2d, src_rows, dst_rows, zeros_hbm)
    return _tc_post2(acc2, degp, W2, b2, n)


# speedup vs baseline: 64.1364x; 1.0428x over previous
import functools

import jax
import jax.numpy as jnp
from jax import lax
from jax.experimental import pallas as pl
from jax.experimental.pallas import tpu as pltpu
from jax.experimental.pallas import tpu_sc as plsc

_NC = 2
_NS = 16
_L = 16
_NW = _NC * _NS
_ROW = 128
_RPW = 80
_K = 8

_E_PAD = _NW * _RPW * _ROW


def _sc_mesh():
    return plsc.VectorSubcoreMesh(core_axis_name="c", subcore_axis_name="s")


_SC_PARAMS = pltpu.CompilerParams(use_tc_tiling_on_sc=False)


def _edge_stream_pipeline(srcv, dstv, rows, hs_s, acc, gsems, ssems):
    n_chunks = _RPW // _K

    def fire_gathers(ch):
        h = ch % 2
        return [
            pltpu.async_copy(hs_s.at[srcv.at[ch * _K + j]],
                             rows.at[h, j], gsems[h])
            for j in range(_K)
        ]

    def fire_scatters(ch):
        h = ch % 2
        return [
            pltpu.async_copy(rows.at[h, j], acc.at[dstv.at[ch * _K + j]],
                             ssems[h], add=True)
            for j in range(_K)
        ]

    gathers = fire_gathers(0)
    scatters = [None] * n_chunks
    for ch in range(n_chunks):
        for cp in gathers:
            cp.wait()
        if ch + 1 < n_chunks:
            if ch >= 1:
                for cp in scatters[ch - 1]:
                    cp.wait()
            gathers = fire_gathers(ch + 1)
        scatters[ch] = fire_scatters(ch)
    for ch in (n_chunks - 2, n_chunks - 1):
        for cp in scatters[ch]:
            cp.wait()


def _sc_rsqrt(x):
    xi = lax.bitcast_convert_type(x, jnp.int32)
    yi = jnp.int32(0x5F3759DF) - lax.shift_right_logical(xi, 1)
    y = lax.bitcast_convert_type(yi, jnp.float32)
    for _ in range(3):
        y = y * (1.5 - 0.5 * x * y * y)
    return y


def _sc_degree(dst_rows, ones_hbm, zeros_hbm, n_pad):
    npt = n_pad // _NS

    @functools.partial(
        pl.kernel,
        out_type=jax.ShapeDtypeStruct((_NC, n_pad, _L), jnp.float32),
        mesh=_sc_mesh(),
        compiler_params=_SC_PARAMS,
        scratch_types=[
            pltpu.VMEM((_RPW, _ROW), jnp.int32),
            pltpu.VMEM((_ROW, _L), jnp.float32),
            pltpu.VMEM_SHARED((n_pad, _L), jnp.float32),
            pltpu.SemaphoreType.DMA,
        ],
    )
    def deg_kernel(dst_hbm, ones_h, zeros_h, out_hbm, dstv, onesv, acc, sem):
        c = lax.axis_index("c")
        s = lax.axis_index("s")
        wid = c * _NS + s
        sl = pl.ds(s * npt, npt)

        stage = [
            pltpu.async_copy(ones_h, onesv, sem),
            pltpu.async_copy(zeros_h, acc.at[sl], sem),
            pltpu.async_copy(dst_hbm.at[wid], dstv, sem),
        ]
        for cp in stage:
            cp.wait()
        plsc.subcore_barrier()

        copies = [
            pltpu.async_copy(onesv, acc.at[dstv.at[r]], sem, add=True)
            for r in range(_RPW)
        ]
        for cp in copies:
            cp.wait()

        plsc.subcore_barrier()
        pltpu.sync_copy(acc.at[sl], out_hbm.at[c, sl])

    return deg_kernel(dst_rows, ones_hbm, zeros_hbm)


def _sc_aggregate1(h1, degp, src_rows, dst_rows, zeros_hbm):
    n_pad = h1.shape[0]
    npt = n_pad // _NS

    @functools.partial(
        pl.kernel,
        out_type=jax.ShapeDtypeStruct((_NC, n_pad, _L), jnp.float32),
        mesh=_sc_mesh(),
        compiler_params=_SC_PARAMS,
        scratch_types=[
            pltpu.VMEM((_RPW, _ROW), jnp.int32),
            pltpu.VMEM((_RPW, _ROW), jnp.int32),
            pltpu.VMEM((2, _K, _ROW, _L), jnp.float32),
            pltpu.VMEM((n_pad // _NS, _L), jnp.float32),
            pltpu.VMEM((n_pad // _NS, _L), jnp.float32),
            pltpu.VMEM((n_pad // _NS, _L), jnp.float32),
            pltpu.VMEM_SHARED((n_pad, _L), jnp.float32),
            pltpu.VMEM_SHARED((n_pad, _L), jnp.float32),
            pltpu.SemaphoreType.DMA,
            pltpu.SemaphoreType.DMA,
            pltpu.SemaphoreType.DMA,
            pltpu.SemaphoreType.DMA,
        ],
    )
    def agg_kernel(h_hbm, d_hbm, src_hbm, dst_hbm, zeros_h, out_hbm,
                   srcv, dstv, rows, hv, d0v, d1v, acc, hs_s,
                   gsem0, gsem1, ssem0, ssem1):
        c = lax.axis_index("c")
        s = lax.axis_index("s")
        wid = c * _NS + s
        sl = pl.ds(s * npt, npt)

        stage = [
            pltpu.async_copy(h_hbm.at[sl], hv, gsem0),
            pltpu.async_copy(d_hbm.at[0, sl], d0v, gsem0),
            pltpu.async_copy(d_hbm.at[1, sl], d1v, gsem0),
        ]
        idx_stage = [
            pltpu.async_copy(src_hbm.at[wid], srcv, gsem1),
            pltpu.async_copy(dst_hbm.at[wid], dstv, gsem1),
        ]
        for cp in stage:
            cp.wait()

        def body(i, carry):
            for j in range(4):
                r = i * 4 + j
                dv = _sc_rsqrt(d0v[r] + d1v[r] + 1.0)
                hv[r] = hv[r] * dv
            return carry

        lax.fori_loop(0, npt // 4, body, 0)

        for cp in idx_stage:
            cp.wait()

        pltpu.sync_copy(hv, hs_s.at[sl])

        @pl.when(c == 0)
        def _():
            pltpu.sync_copy(hv, acc.at[sl])

        @pl.when(c != 0)
        def _():
            pltpu.sync_copy(zeros_h, acc.at[sl])

        plsc.subcore_barrier()

        _edge_stream_pipeline(srcv, dstv, rows, hs_s, acc,
                              (gsem0, gsem1), (ssem0, ssem1))

        plsc.subcore_barrier()
        pltpu.sync_copy(acc.at[sl], out_hbm.at[c, sl])

    return agg_kernel(h1, degp, src_rows, dst_rows, zeros_hbm)


def _sc_aggregate2(degp, acc1, b1_2d, src_rows, dst_rows, zeros_hbm):
    n_pad = acc1.shape[1]
    npt = n_pad // _NS

    @functools.partial(
        pl.kernel,
        out_type=jax.ShapeDtypeStruct((_NC, n_pad, _L), jnp.float32),
        mesh=_sc_mesh(),
        compiler_params=_SC_PARAMS,
        scratch_types=[
            pltpu.VMEM((_RPW, _ROW), jnp.int32),
            pltpu.VMEM((_RPW, _ROW), jnp.int32),
            pltpu.VMEM((2, _K, _ROW, _L), jnp.float32),
            pltpu.VMEM((n_pad // _NS, _L), jnp.float32),
            pltpu.VMEM((n_pad // _NS, _L), jnp.float32),
            pltpu.VMEM((n_pad // _NS, _L), jnp.float32),
            pltpu.VMEM((n_pad // _NS, _L), jnp.float32),
            pltpu.VMEM((n_pad // _NS, _L), jnp.float32),
            pltpu.VMEM((1, _L), jnp.float32),
            pltpu.VMEM_SHARED((n_pad, _L), jnp.float32),
            pltpu.VMEM_SHARED((n_pad, _L), jnp.float32),
            pltpu.SemaphoreType.DMA,
            pltpu.SemaphoreType.DMA,
            pltpu.SemaphoreType.DMA,
            pltpu.SemaphoreType.DMA,
        ],
    )
    def agg2_kernel(d_hbm, a_hbm, b1_hbm, src_hbm, dst_hbm, zeros_h,
                    out_hbm, srcv, dstv, rows, d0v, d1v, a0v, a1v, hsv,
                    b1v, acc, hs_s, gsem0, gsem1, ssem0, ssem1):
        c = lax.axis_index("c")
        s = lax.axis_index("s")
        wid = c * _NS + s
        sl = pl.ds(s * npt, npt)

        stage = [
            pltpu.async_copy(d_hbm.at[0, sl], d0v, gsem0),
            pltpu.async_copy(d_hbm.at[1, sl], d1v, gsem0),
            pltpu.async_copy(a_hbm.at[0, sl], a0v, gsem0),
            pltpu.async_copy(a_hbm.at[1, sl], a1v, gsem0),
            pltpu.async_copy(b1_hbm, b1v, gsem0),
        ]
        idx_stage = [
            pltpu.async_copy(src_hbm.at[wid], srcv, gsem1),
            pltpu.async_copy(dst_hbm.at[wid], dstv, gsem1),
        ]
        for cp in stage:
            cp.wait()

        b = b1v[0]

        def body(i, carry):
            for j in range(4):
                r = i * 4 + j
                dv = _sc_rsqrt(d0v[r] + d1v[r] + 1.0)
                t = (a0v[r] + a1v[r]) * dv + b
                hsv[r] = jnp.maximum(t, 0.0) * dv
            return carry

        lax.fori_loop(0, npt // 4, body, 0)

        for cp in idx_stage:
            cp.wait()

        pltpu.sync_copy(hsv, hs_s.at[sl])

        @pl.when(c == 0)
        def _():
            pltpu.sync_copy(hsv, acc.at[sl])

        @pl.when(c != 0)
        def _():
            pltpu.sync_copy(zeros_h, acc.at[sl])

        plsc.subcore_barrier()

        _edge_stream_pipeline(srcv, dstv, rows, hs_s, acc,
                              (gsem0, gsem1), (ssem0, ssem1))

        plsc.subcore_barrier()
        pltpu.sync_copy(acc.at[sl], out_hbm.at[c, sl])

    return agg2_kernel(degp, acc1, b1_2d, src_rows, dst_rows, zeros_hbm)


def _tc_mm1(x, w, n_pad):
    n, _ = x.shape
    d_out = w.shape[1]

    def body(x_ref, w_ref, o_ref):
        o_ref[...] = jnp.zeros((n_pad, d_out), jnp.float32)
        o_ref[0:n, :] = jnp.dot(x_ref[...], w_ref[...],
                                preferred_element_type=jnp.float32)

    return pl.pallas_call(
        body,
        out_shape=jax.ShapeDtypeStruct((n_pad, d_out), jnp.float32),
    )(x, w)


def _tc_post2(acc, degp, w2, b2, n):
    d_out = w2.shape[1]

    def body(a_ref, d_ref, w_ref, b_ref, o_ref):
        dinv = lax.rsqrt(d_ref[0, 0:n] + d_ref[1, 0:n] + 1.0)
        z = (a_ref[0, 0:n] + a_ref[1, 0:n]) * dinv
        y = jnp.dot(z, w_ref[...], preferred_element_type=jnp.float32)
        y = y + b_ref[...]
        m = jnp.max(y, axis=1, keepdims=True)
        e = jnp.exp(y - m)
        o_ref[...] = (y - m) - jnp.log(jnp.sum(e, axis=1, keepdims=True))

    return pl.pallas_call(
        body,
        out_shape=jax.ShapeDtypeStruct((n, d_out), jnp.float32),
    )(acc, degp, w2, b2)


def kernel(x, edge_index, W1, b1, W2, b2):
    n = x.shape[0]
    e = edge_index.shape[1]
    n_pad = ((n + 1 + _NS * 8 - 1) // (_NS * 8)) * (_NS * 8)
    assert _E_PAD >= e

    src = edge_index[0]
    dst = edge_index[1]
    pad = jnp.full((_E_PAD - e,), n, dtype=jnp.int32)
    row_shape = (_NW, _RPW, _ROW)
    src_rows = jnp.concatenate([src, pad]).reshape(row_shape)
    dst_rows = jnp.concatenate([dst, pad]).reshape(row_shape)
    ones_hbm = jnp.ones((_ROW, _L), jnp.float32)
    zeros_hbm = jnp.zeros((n_pad // _NS, _L), jnp.float32)
    b1_2d = b1.reshape(1, _L)

    degp = _sc_degree(dst_rows, ones_hbm, zeros_hbm, n_pad)
    h1 = _tc_mm1(x, W1, n_pad)
    acc1 = _sc_aggregate1(h1, degp, src_rows, dst_rows, zeros_hbm)
    acc2 = _sc_aggregate2(degp, acc1, b1_2d, src_rows, dst_rows, zeros_hbm)
    return _tc_post2(acc2, degp, W2, b2, n)
